# baseline (speedup 1.0000x reference)
import functools

import jax
import jax.numpy as jnp
from jax import lax
from jax.experimental import pallas as pl
from jax.experimental.pallas import tpu as pltpu
from jax.experimental.pallas import tpu_sc as plsc

_N = 10000
_E = 320000
_C = 128
_NB = 512
_CUTOFF = 5.0
_DELTA = _CUTOFF / (_NB - 1)
_COEFF = -0.5 / (2.0 * _DELTA) ** 2

_BN = 2000
_BE = 512

_NC = 2
_NS = 16
_NW = _NC * _NS
_EW = _E // _NW
_K = 80
_NCHUNK = _EW // _K
_NPAD = 10240
_RPS = _NPAD // _NS
_RB = 160


def _silu(v):
    return v * jax.nn.sigmoid(v)


def _node_embed_body(types_ref, frac_ref, emb_ref, wfrac_ref, wmsg_ref,
                     x_ref, y_ref):
    t = types_ref[...]
    col = lax.broadcasted_iota(jnp.int32, (_BN, _C), 1)
    onehot = (t == col).astype(jnp.float32)
    x = jnp.dot(onehot, emb_ref[...], preferred_element_type=jnp.float32)
    x = x + jnp.dot(frac_ref[...], wfrac_ref[...],
                    preferred_element_type=jnp.float32)
    x_ref[...] = x
    y_ref[...] = jnp.dot(x, wmsg_ref[...], preferred_element_type=jnp.float32)


def _node_embed(types2d, frac, emb_pad, w_frac, w_msg):
    grid = _N // _BN
    return pl.pallas_call(
        _node_embed_body,
        grid=(grid,),
        in_specs=[
            pl.BlockSpec((_BN, 1), lambda i: (i, 0)),
            pl.BlockSpec((_BN, 3), lambda i: (i, 0)),
            pl.BlockSpec((_C, _C), lambda i: (0, 0)),
            pl.BlockSpec((3, _C), lambda i: (0, 0)),
            pl.BlockSpec((_C, _C), lambda i: (0, 0)),
        ],
        out_specs=[
            pl.BlockSpec((_BN, _C), lambda i: (i, 0)),
            pl.BlockSpec((_BN, _C), lambda i: (i, 0)),
        ],
        out_shape=[
            jax.ShapeDtypeStruct((_N, _C), jnp.float32),
            jax.ShapeDtypeStruct((_N, _C), jnp.float32),
        ],
    )(types2d, frac, emb_pad, w_frac, w_msg)


def _edge_mlp_body(d_ref, w1_ref, b1_ref, w2_ref, b2_ref, e_ref):
    d = d_ref[...]
    off = lax.broadcasted_iota(jnp.int32, (_BE, _NB), 1).astype(
        jnp.float32) * _DELTA
    r = d - off
    rbf = jnp.exp(_COEFF * (r * r))
    h = jnp.dot(rbf, w1_ref[...], preferred_element_type=jnp.float32)
    h = _silu(h + b1_ref[...])
    h = jnp.dot(h, w2_ref[...], preferred_element_type=jnp.float32)
    h = _silu(h + b2_ref[...])
    e_ref[...] = h


def _edge_mlp(d2, w1, b1, w2, b2):
    grid = _E // _BE
    return pl.pallas_call(
        _edge_mlp_body,
        grid=(grid,),
        in_specs=[
            pl.BlockSpec((_BE, 1), lambda i: (i, 0)),
            pl.BlockSpec((_NB, _C), lambda i: (0, 0)),
            pl.BlockSpec((1, _C), lambda i: (0, 0)),
            pl.BlockSpec((_C, _C), lambda i: (0, 0)),
            pl.BlockSpec((1, _C), lambda i: (0, 0)),
        ],
        out_specs=pl.BlockSpec((_BE, _C), lambda i: (i, 0)),
        out_shape=jax.ShapeDtypeStruct((_E, _C), jnp.float32),
    )(d2, w1, b1, w2, b2)


def _sc_msg_body(y_hbm, e_hbm, src_hbm, dst_hbm, out_hbm,
                 sidxb, didxb, rowsb, evb, agg_sh,
                 semi, semr, seme, sems):
    cid = lax.axis_index("c")
    sid = lax.axis_index("s")

    def _zrow(i, _):
        for c8 in range(_C // 16):
            rowsb[0, i, pl.ds(c8 * 16, 16)] = jnp.zeros((16,), jnp.float32)
        return 0

    lax.fori_loop(0, _K, _zrow, 0)
    row0 = sid * _RPS
    for t in range(_RPS // _K):
        pltpu.sync_copy(rowsb.at[0], agg_sh.at[pl.ds(row0 + t * _K, _K)])
    plsc.subcore_barrier()

    wid = sid * _NC + cid
    base = wid * _EW

    def _issue_idx(k):
        off = base + k * _K
        s = k % 4
        pltpu.async_copy(src_hbm.at[pl.ds(off, _K)], sidxb.at[s], semi)
        pltpu.async_copy(dst_hbm.at[pl.ds(off, _K)], didxb.at[s], semi)

    def _wait_idx():
        pltpu.make_async_copy(src_hbm.at[pl.ds(0, _K)], sidxb.at[0],
                              semi).wait()
        pltpu.make_async_copy(dst_hbm.at[pl.ds(0, _K)], didxb.at[0],
                              semi).wait()

    def _issue_gather(k, b):
        off = base + k * _K
        pltpu.async_copy(y_hbm.at[sidxb.at[k % 4]], rowsb.at[b], semr)
        pltpu.async_copy(e_hbm.at[pl.ds(off, _K), :], evb.at[b], seme)

    def _wait_gather(b):
        pltpu.make_async_copy(y_hbm.at[pl.ds(0, _K), :], rowsb.at[b],
                              semr).wait()
        pltpu.make_async_copy(e_hbm.at[pl.ds(0, _K), :], evb.at[b],
                              seme).wait()

    _issue_idx(0)
    _issue_idx(1)
    _wait_idx()
    _issue_gather(0, 0)

    def _body(g, _):
        b = lax.rem(g, 2)
        @pl.when(g + 1 < _NCHUNK)
        def _():
            _wait_idx()

        @pl.when(g >= 1)
        def _():
            pltpu.make_async_copy(rowsb.at[0], agg_sh.at[didxb.at[0]],
                                  sems).wait()

        @pl.when(g + 2 < _NCHUNK)
        def _():
            _issue_idx(g + 2)

        @pl.when(g + 1 < _NCHUNK)
        def _():
            _issue_gather(g + 1, 1 - b)

        _wait_gather(b)

        def _mul(r, _):
            for c8 in range(_C // 16):
                sl = pl.ds(c8 * 16, 16)
                rowsb[b, r, sl] = rowsb[b, r, sl] * evb[b, r, sl]
            return 0

        lax.fori_loop(0, _K, _mul, 0)
        pltpu.async_copy(rowsb.at[b], agg_sh.at[didxb.at[g % 4]], sems,
                         add=True)
        return 0

    lax.fori_loop(0, _NCHUNK, _body, 0)
    pltpu.make_async_copy(rowsb.at[0], agg_sh.at[didxb.at[0]], sems).wait()
    plsc.subcore_barrier()

    for t in range(_RPS // _K):
        pltpu.sync_copy(agg_sh.at[pl.ds(row0 + t * _K, _K)], rowsb.at[0])
        pltpu.sync_copy(rowsb.at[0], out_hbm.at[cid, pl.ds(row0 + t * _K, _K)])


def _sc_msg(y, e, src, dst):
    mesh = plsc.VectorSubcoreMesh(core_axis_name="c", subcore_axis_name="s")
    return pl.kernel(
        _sc_msg_body,
        out_type=jax.ShapeDtypeStruct((_NC, _NPAD, _C), jnp.float32),
        mesh=mesh,
        scratch_types=[
            pltpu.VMEM((4, _K), jnp.int32),
            pltpu.VMEM((4, _K), jnp.int32),
            pltpu.VMEM((2, _K, _C), jnp.float32),
            pltpu.VMEM((2, _K, _C), jnp.float32),
            pltpu.VMEM_SHARED((_NPAD, _C), jnp.float32),
            pltpu.SemaphoreType.DMA,
            pltpu.SemaphoreType.DMA,
            pltpu.SemaphoreType.DMA,
            pltpu.SemaphoreType.DMA,
        ],
    )(y, e, src, dst)


def _finalize_body(agg_ref, x_ref, wout_ref, o_ref):
    a = agg_ref[0] + agg_ref[1]
    h = jnp.dot(a, wout_ref[...], preferred_element_type=jnp.float32)
    o_ref[...] = _silu(h) + x_ref[...]


def _finalize(agg, x, w_out):
    grid = _N // _BN
    return pl.pallas_call(
        _finalize_body,
        grid=(grid,),
        in_specs=[
            pl.BlockSpec((_NC, _BN, _C), lambda i: (0, i, 0)),
            pl.BlockSpec((_BN, _C), lambda i: (i, 0)),
            pl.BlockSpec((_C, _C), lambda i: (0, 0)),
        ],
        out_specs=pl.BlockSpec((_BN, _C), lambda i: (i, 0)),
        out_shape=jax.ShapeDtypeStruct((_N, _C), jnp.float32),
    )(agg, x, w_out)


def kernel(atom_types, frac_coords, edge_index, edge_distance, atom_emb,
           W_frac, W_edge1, b_edge1, W_edge2, b_edge2, W_msg, W_out):
    types2d = atom_types.astype(jnp.int32).reshape(_N, 1)
    emb_pad = jnp.zeros((_C, _C), jnp.float32).at[:atom_emb.shape[0]].set(
        atom_emb)
    x, y = _node_embed(types2d, frac_coords, emb_pad, W_frac, W_msg)

    e = _edge_mlp(edge_distance.reshape(_E, 1), W_edge1,
                  b_edge1.reshape(1, _C), W_edge2, b_edge2.reshape(1, _C))

    src = edge_index[0].astype(jnp.int32)
    dst = edge_index[1].astype(jnp.int32)
    agg = _sc_msg(y, e, src, dst)

    return _finalize(agg[:, :_N, :], x, W_out)


# speedup vs baseline: 1.3527x; 1.3527x over previous
import functools

import jax
import jax.numpy as jnp
from jax import lax
from jax.experimental import pallas as pl
from jax.experimental.pallas import tpu as pltpu
from jax.experimental.pallas import tpu_sc as plsc

_N = 10000
_E = 320000
_C = 128
_NB = 512
_CUTOFF = 5.0
_DELTA = _CUTOFF / (_NB - 1)
_COEFF = -0.5 / (2.0 * _DELTA) ** 2

_BN = 2000
_BE = 512

_NC = 2
_NS = 16
_NW = _NC * _NS
_EW = _E // _NW
_K = 80
_NCHUNK = _EW // _K
_NPAD = 10240
_RPS = _NPAD // _NS
_RB = 160


def _silu(v):
    return v * jax.nn.sigmoid(v)


def _node_embed_body(types_ref, frac_ref, emb_ref, wfrac_ref, wmsg_ref,
                     x_ref, y_ref):
    t = types_ref[...]
    col = lax.broadcasted_iota(jnp.int32, (_BN, _C), 1)
    onehot = (t == col).astype(jnp.float32)
    x = jnp.dot(onehot, emb_ref[...], preferred_element_type=jnp.float32)
    x = x + jnp.dot(frac_ref[...], wfrac_ref[...],
                    preferred_element_type=jnp.float32)
    x_ref[...] = x
    y_ref[...] = jnp.dot(x, wmsg_ref[...], preferred_element_type=jnp.float32)


def _node_embed(types2d, frac, emb_pad, w_frac, w_msg):
    grid = _N // _BN
    return pl.pallas_call(
        _node_embed_body,
        grid=(grid,),
        in_specs=[
            pl.BlockSpec((_BN, 1), lambda i: (i, 0)),
            pl.BlockSpec((_BN, 3), lambda i: (i, 0)),
            pl.BlockSpec((_C, _C), lambda i: (0, 0)),
            pl.BlockSpec((3, _C), lambda i: (0, 0)),
            pl.BlockSpec((_C, _C), lambda i: (0, 0)),
        ],
        out_specs=[
            pl.BlockSpec((_BN, _C), lambda i: (i, 0)),
            pl.BlockSpec((_BN, _C), lambda i: (i, 0)),
        ],
        out_shape=[
            jax.ShapeDtypeStruct((_N, _C), jnp.float32),
            jax.ShapeDtypeStruct((_N, _C), jnp.float32),
        ],
    )(types2d, frac, emb_pad, w_frac, w_msg)


def _edge_mlp_body(d_ref, w1_ref, b1_ref, w2_ref, b2_ref, e_ref):
    d = d_ref[...]
    off = lax.broadcasted_iota(jnp.int32, (_BE, _NB), 1).astype(
        jnp.float32) * _DELTA
    r = d - off
    rbf = jnp.exp(_COEFF * (r * r))
    h = jnp.dot(rbf, w1_ref[...], preferred_element_type=jnp.float32)
    h = _silu(h + b1_ref[...])
    h = jnp.dot(h, w2_ref[...], preferred_element_type=jnp.float32)
    h = _silu(h + b2_ref[...])
    e_ref[...] = h


def _edge_mlp(d2, w1, b1, w2, b2):
    grid = _E // _BE
    return pl.pallas_call(
        _edge_mlp_body,
        grid=(grid,),
        in_specs=[
            pl.BlockSpec((_BE, 1), lambda i: (i, 0)),
            pl.BlockSpec((_NB, _C), lambda i: (0, 0)),
            pl.BlockSpec((1, _C), lambda i: (0, 0)),
            pl.BlockSpec((_C, _C), lambda i: (0, 0)),
            pl.BlockSpec((1, _C), lambda i: (0, 0)),
        ],
        out_specs=pl.BlockSpec((_BE, _C), lambda i: (i, 0)),
        out_shape=jax.ShapeDtypeStruct((_E, _C), jnp.float32),
    )(d2, w1, b1, w2, b2)


def _sc_msg_body(y_hbm, e_hbm, src_hbm, dst_hbm, out_hbm,
                 sidxb, didxb, rowsb, evb, agg_sh,
                 semi, semr, seme, sems):
    cid = lax.axis_index("c")
    sid = lax.axis_index("s")

    def _zrow(i, _):
        for c8 in range(_C // 16):
            rowsb[0, i, pl.ds(c8 * 16, 16)] = jnp.zeros((16,), jnp.float32)
        return 0

    lax.fori_loop(0, _K, _zrow, 0)
    row0 = sid * _RPS
    for t in range(_RPS // _K):
        pltpu.sync_copy(rowsb.at[0], agg_sh.at[pl.ds(row0 + t * _K, _K)])
    plsc.subcore_barrier()

    wid = sid * _NC + cid
    base = wid * _EW

    def _issue_idx(k):
        off = base + k * _K
        s = k % 4
        pltpu.async_copy(src_hbm.at[pl.ds(off, _K)], sidxb.at[s], semi)
        pltpu.async_copy(dst_hbm.at[pl.ds(off, _K)], didxb.at[s], semi)

    def _wait_idx():
        pltpu.make_async_copy(src_hbm.at[pl.ds(0, _K)], sidxb.at[0],
                              semi).wait()
        pltpu.make_async_copy(dst_hbm.at[pl.ds(0, _K)], didxb.at[0],
                              semi).wait()

    def _issue_gather(k, b):
        off = base + k * _K
        pltpu.async_copy(y_hbm.at[sidxb.at[k % 4]], rowsb.at[b], semr)
        pltpu.async_copy(e_hbm.at[pl.ds(off, _K), :], evb.at[b], seme)

    def _wait_gather(b):
        pltpu.make_async_copy(y_hbm.at[pl.ds(0, _K), :], rowsb.at[b],
                              semr).wait()
        pltpu.make_async_copy(e_hbm.at[pl.ds(0, _K), :], evb.at[b],
                              seme).wait()

    _issue_idx(0)
    _issue_idx(1)
    _wait_idx()
    _issue_gather(0, 0)

    def _body(g, _):
        b = lax.rem(g, 2)
        @pl.when(g + 1 < _NCHUNK)
        def _():
            _wait_idx()

        @pl.when(g >= 1)
        def _():
            pltpu.make_async_copy(rowsb.at[0], agg_sh.at[didxb.at[0]],
                                  sems).wait()

        @pl.when(g + 2 < _NCHUNK)
        def _():
            _issue_idx(g + 2)

        @pl.when(g + 1 < _NCHUNK)
        def _():
            _issue_gather(g + 1, 1 - b)

        _wait_gather(b)

        def _do_mul(bs):
            @plsc.parallel_loop(0, _K, 1, unroll=8)
            def _mul(r):
                for c8 in range(_C // 16):
                    sl = pl.ds(c8 * 16, 16)
                    rowsb[bs, r, sl] = rowsb[bs, r, sl] * evb[bs, r, sl]

        @pl.when(b == 0)
        def _():
            _do_mul(0)

        @pl.when(b == 1)
        def _():
            _do_mul(1)
        pltpu.async_copy(rowsb.at[b], agg_sh.at[didxb.at[g % 4]], sems,
                         add=True)
        return 0

    lax.fori_loop(0, _NCHUNK, _body, 0)
    pltpu.make_async_copy(rowsb.at[0], agg_sh.at[didxb.at[0]], sems).wait()
    plsc.subcore_barrier()

    for t in range(_RPS // _K):
        pltpu.sync_copy(agg_sh.at[pl.ds(row0 + t * _K, _K)], rowsb.at[0])
        pltpu.sync_copy(rowsb.at[0], out_hbm.at[cid, pl.ds(row0 + t * _K, _K)])


def _sc_msg(y, e, src, dst):
    mesh = plsc.VectorSubcoreMesh(core_axis_name="c", subcore_axis_name="s")
    return pl.kernel(
        _sc_msg_body,
        out_type=jax.ShapeDtypeStruct((_NC, _NPAD, _C), jnp.float32),
        mesh=mesh,
        scratch_types=[
            pltpu.VMEM((4, _K), jnp.int32),
            pltpu.VMEM((4, _K), jnp.int32),
            pltpu.VMEM((2, _K, _C), jnp.float32),
            pltpu.VMEM((2, _K, _C), jnp.float32),
            pltpu.VMEM_SHARED((_NPAD, _C), jnp.float32),
            pltpu.SemaphoreType.DMA,
            pltpu.SemaphoreType.DMA,
            pltpu.SemaphoreType.DMA,
            pltpu.SemaphoreType.DMA,
        ],
    )(y, e, src, dst)


def _finalize_body(agg_ref, x_ref, wout_ref, o_ref):
    a = agg_ref[0] + agg_ref[1]
    h = jnp.dot(a, wout_ref[...], preferred_element_type=jnp.float32)
    o_ref[...] = _silu(h) + x_ref[...]


def _finalize(agg, x, w_out):
    grid = _N // _BN
    return pl.pallas_call(
        _finalize_body,
        grid=(grid,),
        in_specs=[
            pl.BlockSpec((_NC, _BN, _C), lambda i: (0, i, 0)),
            pl.BlockSpec((_BN, _C), lambda i: (i, 0)),
            pl.BlockSpec((_C, _C), lambda i: (0, 0)),
        ],
        out_specs=pl.BlockSpec((_BN, _C), lambda i: (i, 0)),
        out_shape=jax.ShapeDtypeStruct((_N, _C), jnp.float32),
    )(agg, x, w_out)


def kernel(atom_types, frac_coords, edge_index, edge_distance, atom_emb,
           W_frac, W_edge1, b_edge1, W_edge2, b_edge2, W_msg, W_out):
    types2d = atom_types.astype(jnp.int32).reshape(_N, 1)
    emb_pad = jnp.zeros((_C, _C), jnp.float32).at[:atom_emb.shape[0]].set(
        atom_emb)
    x, y = _node_embed(types2d, frac_coords, emb_pad, W_frac, W_msg)

    e = _edge_mlp(edge_distance.reshape(_E, 1), W_edge1,
                  b_edge1.reshape(1, _C), W_edge2, b_edge2.reshape(1, _C))

    src = edge_index[0].astype(jnp.int32)
    dst = edge_index[1].astype(jnp.int32)
    agg = _sc_msg(y, e, src, dst)

    return _finalize(agg[:, :_N, :], x, W_out)
